# Initial kernel scaffold; baseline (speedup 1.0000x reference)
import jax
import jax.numpy as jnp
from jax.experimental import pallas as pl


def kernel(x, edge_index, W1, b1, W2, b2):
    raise NotImplementedError("write your pallas kernel here")


# speedup vs baseline: 19.9249x; 19.9249x over previous
import functools

import jax
import jax.numpy as jnp
from jax import lax
from jax.experimental import pallas as pl
from jax.experimental.pallas import tpu as pltpu
from jax.experimental.pallas import tpu_sc as plsc

NC = 2
NS = 16
NW = NC * NS

CH = 128
BG = 4


def _sc_mesh():
    return plsc.VectorSubcoreMesh(
        core_axis_name="c", subcore_axis_name="s", num_cores=NC, num_subcores=NS
    )


_SC_PARAMS = pltpu.CompilerParams(use_tc_tiling_on_sc=False)


def _make_deg_kernel(n_acc, cpt, dw):
    rows_per_tile = n_acc // NS

    @functools.partial(
        pl.kernel,
        out_type=jax.ShapeDtypeStruct((NC, n_acc, dw), jnp.float32),
        mesh=_sc_mesh(),
        compiler_params=_SC_PARAMS,
        scratch_types=[
            pltpu.VMEM((cpt, CH), jnp.int32),
            pltpu.VMEM((CH, dw), jnp.float32),
            pltpu.VMEM_SHARED((n_acc, dw), jnp.float32),
            pltpu.SemaphoreType.DMA,
        ],
    )
    def k(dst_hbm, ones_hbm, zinit_hbm, out_hbm, dst_v, ones_v, acc_sh, sem):
        c = lax.axis_index("c")
        s = lax.axis_index("s")
        w = c * NS + s
        pltpu.sync_copy(dst_hbm.at[w], dst_v)
        pltpu.sync_copy(ones_hbm, ones_v)
        pltpu.sync_copy(
            zinit_hbm.at[pl.ds(s * rows_per_tile, rows_per_tile)],
            acc_sh.at[pl.ds(s * rows_per_tile, rows_per_tile)],
        )
        plsc.subcore_barrier()

        def fire(j, carry):
            pltpu.async_copy(ones_v, acc_sh.at[dst_v.at[j]], sem, add=True)
            return carry

        lax.fori_loop(0, cpt, fire, 0)

        def drain(j, carry):
            pltpu.make_async_copy(ones_v, acc_sh.at[dst_v.at[j]], sem).wait()
            return carry

        lax.fori_loop(0, cpt, drain, 0)
        plsc.subcore_barrier()
        pltpu.sync_copy(
            acc_sh.at[pl.ds(s * rows_per_tile, rows_per_tile)],
            out_hbm.at[c, pl.ds(s * rows_per_tile, rows_per_tile)],
        )

    return k


def _make_scatter_kernel(n_acc, cpt, width):
    rows_per_tile = n_acc // NS
    nb = cpt // BG
    nbp = nb // 2
    assert cpt % (2 * BG) == 0

    @functools.partial(
        pl.kernel,
        out_type=jax.ShapeDtypeStruct((NC, n_acc, width), jnp.float32),
        mesh=_sc_mesh(),
        compiler_params=_SC_PARAMS,
        scratch_types=[
            pltpu.VMEM((cpt, CH), jnp.int32),
            pltpu.VMEM((cpt, CH), jnp.int32),
            pltpu.VMEM((2, BG * CH, width), jnp.float32),
            pltpu.VMEM_SHARED((n_acc, width), jnp.float32),
            pltpu.SemaphoreType.DMA,
            pltpu.SemaphoreType.DMA,
            pltpu.SemaphoreType.DMA,
            pltpu.SemaphoreType.DMA,
        ],
    )
    def k(y_hbm, src_hbm, dst_hbm, zinit_hbm, out_hbm, src_v, dst_v, rows_v,
          acc_sh, semg0, semg1, sems0, sems1):
        c = lax.axis_index("c")
        s = lax.axis_index("s")
        w = c * NS + s
        pltpu.sync_copy(src_hbm.at[w], src_v)
        pltpu.sync_copy(dst_hbm.at[w], dst_v)
        pltpu.sync_copy(
            zinit_hbm.at[pl.ds(s * rows_per_tile, rows_per_tile)],
            acc_sh.at[pl.ds(s * rows_per_tile, rows_per_tile)],
        )
        plsc.subcore_barrier()

        semg = (semg0, semg1)
        sems = (sems0, sems1)

        def gslice(h, i):
            return rows_v.at[h, pl.ds(i * CH, CH)]

        def issue_gathers(b, h):
            for i in range(BG):
                pltpu.async_copy(
                    y_hbm.at[src_v.at[b * BG + i]], gslice(h, i), semg[h]
                )

        def wait_gathers(b, h):
            for i in range(BG):
                pltpu.make_async_copy(
                    y_hbm.at[src_v.at[b * BG + i]], gslice(h, i), semg[h]
                ).wait()

        def issue_scatters(b, h):
            for i in range(BG):
                pltpu.async_copy(
                    gslice(h, i), acc_sh.at[dst_v.at[b * BG + i]],
                    sems[h], add=True,
                )

        def wait_scatters(b, h):
            for i in range(BG):
                pltpu.make_async_copy(
                    gslice(h, i), acc_sh.at[dst_v.at[b * BG + i]], sems[h]
                ).wait()

        issue_gathers(0, 0)

        def body(bp, carry):
            b0 = 2 * bp
            wait_gathers(b0, 0)
            issue_scatters(b0, 0)

            @pl.when(bp > 0)
            def _():
                wait_scatters(b0 - 1, 1)

            issue_gathers(b0 + 1, 1)
            wait_gathers(b0 + 1, 1)
            issue_scatters(b0 + 1, 1)

            @pl.when(bp < nbp - 1)
            def _():
                wait_scatters(b0, 0)
                issue_gathers(b0 + 2, 0)

            return carry

        lax.fori_loop(0, nbp, body, 0)
        wait_scatters(nb - 2, 0)
        wait_scatters(nb - 1, 1)
        plsc.subcore_barrier()
        pltpu.sync_copy(
            acc_sh.at[pl.ds(s * rows_per_tile, rows_per_tile)],
            out_hbm.at[c, pl.ds(s * rows_per_tile, rows_per_tile)],
        )

    return k


def _tc1_body(degacc_ref, x_ref, w1_ref, dinv_ref, y1_ref):
    d = degacc_ref[...]
    deg = d[0, :, 0:1] + d[1, :, 0:1] + 1.0
    dinv = lax.rsqrt(deg)
    xw = jnp.dot(x_ref[...], w1_ref[...], preferred_element_type=jnp.float32)
    y1_ref[...] = xw * dinv
    dinv_ref[...] = jnp.broadcast_to(dinv, dinv_ref.shape)


def _tc2_body(s1_ref, y1_ref, dinv_ref, w2_ref, b1_ref, y2_ref):
    s1 = s1_ref[...]
    dinv = dinv_ref[...][:, 0:1]
    h = (s1[0] + s1[1] + y1_ref[...]) * dinv + b1_ref[...]
    h = jnp.maximum(h, 0.0)
    hw = jnp.dot(h, w2_ref[...], preferred_element_type=jnp.float32)
    y2_ref[...] = hw * dinv


def _tc3_body(s2_ref, y2_ref, dinv_ref, b2_ref, out_ref):
    s2 = s2_ref[...]
    dinv = dinv_ref[...][:, 0:1]
    z = (s2[0] + s2[1] + y2_ref[...]) * dinv + b2_ref[...]
    m = jnp.max(z, axis=1, keepdims=True)
    e = jnp.exp(z - m)
    out_ref[...] = z - m - jnp.log(jnp.sum(e, axis=1, keepdims=True))


def kernel(x, edge_index, W1, b1, W2, b2):
    n, d_in = x.shape
    e = edge_index.shape[1]
    h_dim = W1.shape[1]
    c_dim = W2.shape[1]

    n_acc = ((n + 511) // 512 + 1) * 512
    blk = 512
    n_blocks = n_acc // blk

    ept = -(-e // NW)
    cpt = -(-ept // (CH * 2 * BG)) * 2 * BG
    e_pad = NW * cpt * CH
    n_dummy = n_acc - n
    pad_dst = n + jnp.arange(e_pad - e, dtype=jnp.int32) % n_dummy
    src_p = jnp.concatenate(
        [edge_index[0], jnp.zeros((e_pad - e,), jnp.int32)]
    ).reshape(NW, cpt, CH)
    dst_p = jnp.concatenate([edge_index[1], pad_dst]).reshape(NW, cpt, CH)

    dw = 16
    ones_buf = jnp.ones((CH, dw), jnp.float32)
    zin_dw = jnp.zeros((n_acc, dw), jnp.float32)
    zin_h = jnp.zeros((n_acc, h_dim), jnp.float32)
    zin_c = jnp.zeros((n_acc, c_dim), jnp.float32)
    x_pad = jnp.concatenate([x, jnp.zeros((n_acc - n, d_in), x.dtype)])

    degacc = _make_deg_kernel(n_acc, cpt, dw)(dst_p, ones_buf, zin_dw)

    dinv, y1 = pl.pallas_call(
        _tc1_body,
        grid=(n_blocks,),
        in_specs=[
            pl.BlockSpec((NC, blk, dw), lambda i: (0, i, 0)),
            pl.BlockSpec((blk, d_in), lambda i: (i, 0)),
            pl.BlockSpec((d_in, h_dim), lambda i: (0, 0)),
        ],
        out_specs=[
            pl.BlockSpec((blk, 8), lambda i: (i, 0)),
            pl.BlockSpec((blk, h_dim), lambda i: (i, 0)),
        ],
        out_shape=[
            jax.ShapeDtypeStruct((n_acc, 8), jnp.float32),
            jax.ShapeDtypeStruct((n_acc, h_dim), jnp.float32),
        ],
    )(degacc, x_pad, W1)

    s1 = _make_scatter_kernel(n_acc, cpt, h_dim)(y1, src_p, dst_p, zin_h)

    y2 = pl.pallas_call(
        _tc2_body,
        grid=(n_blocks,),
        in_specs=[
            pl.BlockSpec((NC, blk, h_dim), lambda i: (0, i, 0)),
            pl.BlockSpec((blk, h_dim), lambda i: (i, 0)),
            pl.BlockSpec((blk, 8), lambda i: (i, 0)),
            pl.BlockSpec((h_dim, c_dim), lambda i: (0, 0)),
            pl.BlockSpec((1, h_dim), lambda i: (0, 0)),
        ],
        out_specs=pl.BlockSpec((blk, c_dim), lambda i: (i, 0)),
        out_shape=jax.ShapeDtypeStruct((n_acc, c_dim), jnp.float32),
    )(s1, y1, dinv, W2, b1.reshape(1, h_dim))

    s2 = _make_scatter_kernel(n_acc, cpt, c_dim)(y2, src_p, dst_p, zin_c)

    out = pl.pallas_call(
        _tc3_body,
        grid=(n_blocks,),
        in_specs=[
            pl.BlockSpec((NC, blk, c_dim), lambda i: (0, i, 0)),
            pl.BlockSpec((blk, c_dim), lambda i: (i, 0)),
            pl.BlockSpec((blk, 8), lambda i: (i, 0)),
            pl.BlockSpec((1, c_dim), lambda i: (0, 0)),
        ],
        out_specs=pl.BlockSpec((blk, c_dim), lambda i: (i, 0)),
        out_shape=jax.ShapeDtypeStruct((n_acc, c_dim), jnp.float32),
    )(s2, y2, dinv, b2.reshape(1, c_dim))

    return out[:n]


# speedup vs baseline: 21.8688x; 1.0976x over previous
import functools

import jax
import jax.numpy as jnp
from jax import lax
from jax.experimental import pallas as pl
from jax.experimental.pallas import tpu as pltpu
from jax.experimental.pallas import tpu_sc as plsc

NC = 2
NS = 16
NW = NC * NS

CH = 128
BG = 4


def _sc_mesh():
    return plsc.VectorSubcoreMesh(
        core_axis_name="c", subcore_axis_name="s", num_cores=NC, num_subcores=NS
    )


_SC_PARAMS = pltpu.CompilerParams(use_tc_tiling_on_sc=False)


def _make_deg_kernel(n_acc, cpt, dw):
    rows_per_tile = n_acc // NS

    @functools.partial(
        pl.kernel,
        out_type=jax.ShapeDtypeStruct((NC, n_acc, dw), jnp.float32),
        mesh=_sc_mesh(),
        compiler_params=_SC_PARAMS,
        scratch_types=[
            pltpu.VMEM((cpt, CH), jnp.int32),
            pltpu.VMEM((CH, dw), jnp.float32),
            pltpu.VMEM_SHARED((n_acc, dw), jnp.float32),
            pltpu.SemaphoreType.DMA,
        ],
    )
    def k(dst_hbm, ones_hbm, zinit_hbm, out_hbm, dst_v, ones_v, acc_sh, sem):
        c = lax.axis_index("c")
        s = lax.axis_index("s")
        w = c * NS + s
        pltpu.sync_copy(dst_hbm.at[w], dst_v)
        pltpu.sync_copy(ones_hbm, ones_v)
        pltpu.sync_copy(
            zinit_hbm.at[pl.ds(s * rows_per_tile, rows_per_tile)],
            acc_sh.at[pl.ds(s * rows_per_tile, rows_per_tile)],
        )
        plsc.subcore_barrier()

        def fire(j, carry):
            pltpu.async_copy(ones_v, acc_sh.at[dst_v.at[j]], sem, add=True)
            return carry

        lax.fori_loop(0, cpt, fire, 0)

        def drain(j, carry):
            pltpu.make_async_copy(ones_v, acc_sh.at[dst_v.at[j]], sem).wait()
            return carry

        lax.fori_loop(0, cpt, drain, 0)
        plsc.subcore_barrier()
        pltpu.sync_copy(
            acc_sh.at[pl.ds(s * rows_per_tile, rows_per_tile)],
            out_hbm.at[c, pl.ds(s * rows_per_tile, rows_per_tile)],
        )

    return k


def _make_scatter_kernel(n_acc, cpt, width, stage_y=False):
    rows_per_tile = n_acc // NS
    nb = cpt // BG
    nbp = nb // 2
    assert cpt % (2 * BG) == 0

    @functools.partial(
        pl.kernel,
        out_type=jax.ShapeDtypeStruct((NC, n_acc, width), jnp.float32),
        mesh=_sc_mesh(),
        compiler_params=_SC_PARAMS,
        scratch_types=[
            pltpu.VMEM((cpt, CH), jnp.int32),
            pltpu.VMEM((cpt, CH), jnp.int32),
            pltpu.VMEM((2, BG * CH, width), jnp.float32),
            pltpu.VMEM_SHARED((n_acc, width), jnp.float32),
        ]
        + ([pltpu.VMEM_SHARED((n_acc, width), jnp.float32)] if stage_y else [])
        + [
            pltpu.SemaphoreType.DMA,
            pltpu.SemaphoreType.DMA,
            pltpu.SemaphoreType.DMA,
            pltpu.SemaphoreType.DMA,
        ],
    )
    def k(y_hbm, src_hbm, dst_hbm, zinit_hbm, out_hbm, src_v, dst_v, rows_v,
          acc_sh, *rest):
        if stage_y:
            y_sh, semg0, semg1, sems0, sems1 = rest
        else:
            semg0, semg1, sems0, sems1 = rest
            y_sh = None
        c = lax.axis_index("c")
        s = lax.axis_index("s")
        w = c * NS + s
        pltpu.sync_copy(src_hbm.at[w], src_v)
        pltpu.sync_copy(dst_hbm.at[w], dst_v)
        if stage_y:
            pltpu.sync_copy(
                y_hbm.at[pl.ds(s * rows_per_tile, rows_per_tile)],
                y_sh.at[pl.ds(s * rows_per_tile, rows_per_tile)],
            )
        y_src = y_sh if stage_y else y_hbm
        pltpu.sync_copy(
            zinit_hbm.at[pl.ds(s * rows_per_tile, rows_per_tile)],
            acc_sh.at[pl.ds(s * rows_per_tile, rows_per_tile)],
        )
        plsc.subcore_barrier()

        semg = (semg0, semg1)
        sems = (sems0, sems1)

        def gslice(h, i):
            return rows_v.at[h, pl.ds(i * CH, CH)]

        def issue_gathers(b, h):
            for i in range(BG):
                pltpu.async_copy(
                    y_src.at[src_v.at[b * BG + i]], gslice(h, i), semg[h]
                )

        def wait_gathers(b, h):
            for i in range(BG):
                pltpu.make_async_copy(
                    y_src.at[src_v.at[b * BG + i]], gslice(h, i), semg[h]
                ).wait()

        def issue_scatters(b, h):
            for i in range(BG):
                pltpu.async_copy(
                    gslice(h, i), acc_sh.at[dst_v.at[b * BG + i]],
                    sems[h], add=True,
                )

        def wait_scatters(b, h):
            for i in range(BG):
                pltpu.make_async_copy(
                    gslice(h, i), acc_sh.at[dst_v.at[b * BG + i]], sems[h]
                ).wait()

        issue_gathers(0, 0)

        def body(bp, carry):
            b0 = 2 * bp
            wait_gathers(b0, 0)
            issue_scatters(b0, 0)

            @pl.when(bp > 0)
            def _():
                wait_scatters(b0 - 1, 1)

            issue_gathers(b0 + 1, 1)
            wait_gathers(b0 + 1, 1)
            issue_scatters(b0 + 1, 1)

            @pl.when(bp < nbp - 1)
            def _():
                wait_scatters(b0, 0)
                issue_gathers(b0 + 2, 0)

            return carry

        lax.fori_loop(0, nbp, body, 0)
        wait_scatters(nb - 2, 0)
        wait_scatters(nb - 1, 1)
        plsc.subcore_barrier()
        pltpu.sync_copy(
            acc_sh.at[pl.ds(s * rows_per_tile, rows_per_tile)],
            out_hbm.at[c, pl.ds(s * rows_per_tile, rows_per_tile)],
        )

    return k


def _tc1_body(degacc_ref, x_ref, w1_ref, dinv_ref, y1_ref):
    d = degacc_ref[...]
    deg = d[0, :, 0:1] + d[1, :, 0:1] + 1.0
    dinv = lax.rsqrt(deg)
    xw = jnp.dot(x_ref[...], w1_ref[...], preferred_element_type=jnp.float32)
    y1_ref[...] = xw * dinv
    dinv_ref[...] = jnp.broadcast_to(dinv, dinv_ref.shape)


def _tc2_body(s1_ref, y1_ref, dinv_ref, w2_ref, b1_ref, y2_ref):
    s1 = s1_ref[...]
    dinv = dinv_ref[...][:, 0:1]
    h = (s1[0] + s1[1] + y1_ref[...]) * dinv + b1_ref[...]
    h = jnp.maximum(h, 0.0)
    hw = jnp.dot(h, w2_ref[...], preferred_element_type=jnp.float32)
    y2_ref[...] = hw * dinv


def _tc3_body(s2_ref, y2_ref, dinv_ref, b2_ref, out_ref):
    s2 = s2_ref[...]
    dinv = dinv_ref[...][:, 0:1]
    z = (s2[0] + s2[1] + y2_ref[...]) * dinv + b2_ref[...]
    m = jnp.max(z, axis=1, keepdims=True)
    e = jnp.exp(z - m)
    out_ref[...] = z - m - jnp.log(jnp.sum(e, axis=1, keepdims=True))


def kernel(x, edge_index, W1, b1, W2, b2):
    n, d_in = x.shape
    e = edge_index.shape[1]
    h_dim = W1.shape[1]
    c_dim = W2.shape[1]

    n_acc = ((n + 511) // 512 + 1) * 512
    blk = 512
    n_blocks = n_acc // blk

    ept = -(-e // NW)
    cpt = -(-ept // (CH * 2 * BG)) * 2 * BG
    e_pad = NW * cpt * CH
    n_dummy = n_acc - n
    pad_dst = n + jnp.arange(e_pad - e, dtype=jnp.int32) % n_dummy
    src_p = jnp.concatenate(
        [edge_index[0], jnp.zeros((e_pad - e,), jnp.int32)]
    ).reshape(NW, cpt, CH)
    dst_p = jnp.concatenate([edge_index[1], pad_dst]).reshape(NW, cpt, CH)

    dw = 16
    ones_buf = jnp.ones((CH, dw), jnp.float32)
    zin_dw = jnp.zeros((n_acc, dw), jnp.float32)
    zin_h = jnp.zeros((n_acc, h_dim), jnp.float32)
    zin_c = jnp.zeros((n_acc, c_dim), jnp.float32)
    x_pad = jnp.concatenate([x, jnp.zeros((n_acc - n, d_in), x.dtype)])

    degacc = _make_deg_kernel(n_acc, cpt, dw)(dst_p, ones_buf, zin_dw)

    dinv, y1 = pl.pallas_call(
        _tc1_body,
        grid=(n_blocks,),
        in_specs=[
            pl.BlockSpec((NC, blk, dw), lambda i: (0, i, 0)),
            pl.BlockSpec((blk, d_in), lambda i: (i, 0)),
            pl.BlockSpec((d_in, h_dim), lambda i: (0, 0)),
        ],
        out_specs=[
            pl.BlockSpec((blk, 8), lambda i: (i, 0)),
            pl.BlockSpec((blk, h_dim), lambda i: (i, 0)),
        ],
        out_shape=[
            jax.ShapeDtypeStruct((n_acc, 8), jnp.float32),
            jax.ShapeDtypeStruct((n_acc, h_dim), jnp.float32),
        ],
    )(degacc, x_pad, W1)

    s1 = _make_scatter_kernel(n_acc, cpt, h_dim)(y1, src_p, dst_p, zin_h)

    y2 = pl.pallas_call(
        _tc2_body,
        grid=(n_blocks,),
        in_specs=[
            pl.BlockSpec((NC, blk, h_dim), lambda i: (0, i, 0)),
            pl.BlockSpec((blk, h_dim), lambda i: (i, 0)),
            pl.BlockSpec((blk, 8), lambda i: (i, 0)),
            pl.BlockSpec((h_dim, c_dim), lambda i: (0, 0)),
            pl.BlockSpec((1, h_dim), lambda i: (0, 0)),
        ],
        out_specs=pl.BlockSpec((blk, c_dim), lambda i: (i, 0)),
        out_shape=jax.ShapeDtypeStruct((n_acc, c_dim), jnp.float32),
    )(s1, y1, dinv, W2, b1.reshape(1, h_dim))

    s2 = _make_scatter_kernel(n_acc, cpt, c_dim, stage_y=True)(
        y2, src_p, dst_p, zin_c
    )

    out = pl.pallas_call(
        _tc3_body,
        grid=(n_blocks,),
        in_specs=[
            pl.BlockSpec((NC, blk, c_dim), lambda i: (0, i, 0)),
            pl.BlockSpec((blk, c_dim), lambda i: (i, 0)),
            pl.BlockSpec((blk, 8), lambda i: (i, 0)),
            pl.BlockSpec((1, c_dim), lambda i: (0, 0)),
        ],
        out_specs=pl.BlockSpec((blk, c_dim), lambda i: (i, 0)),
        out_shape=jax.ShapeDtypeStruct((n_acc, c_dim), jnp.float32),
    )(s2, y2, dinv, b2.reshape(1, c_dim))

    return out[:n]


# speedup vs baseline: 39.5920x; 1.8104x over previous
import functools

import jax
import jax.numpy as jnp
from jax import lax
from jax.experimental import pallas as pl
from jax.experimental.pallas import tpu as pltpu
from jax.experimental.pallas import tpu_sc as plsc

NC = 2
NS = 16
NW = NC * NS

CH = 128
BG = 4


def _sc_mesh():
    return plsc.VectorSubcoreMesh(
        core_axis_name="c", subcore_axis_name="s", num_cores=NC, num_subcores=NS
    )


_SC_PARAMS = pltpu.CompilerParams(use_tc_tiling_on_sc=False)


def _make_deg_kernel(n_acc, cpt, dw):
    rows_per_tile = n_acc // NS

    @functools.partial(
        pl.kernel,
        out_type=jax.ShapeDtypeStruct((NC, n_acc, dw), jnp.float32),
        mesh=_sc_mesh(),
        compiler_params=_SC_PARAMS,
        scratch_types=[
            pltpu.VMEM((cpt, CH), jnp.int32),
            pltpu.VMEM((CH, dw), jnp.float32),
            pltpu.VMEM_SHARED((n_acc, dw), jnp.float32),
            pltpu.SemaphoreType.DMA,
        ],
    )
    def k(dst_hbm, ones_hbm, zinit_hbm, out_hbm, dst_v, ones_v, acc_sh, sem):
        c = lax.axis_index("c")
        s = lax.axis_index("s")
        w = c * NS + s
        pltpu.sync_copy(dst_hbm.at[w], dst_v)
        pltpu.sync_copy(ones_hbm, ones_v)
        pltpu.sync_copy(
            zinit_hbm.at[pl.ds(s * rows_per_tile, rows_per_tile)],
            acc_sh.at[pl.ds(s * rows_per_tile, rows_per_tile)],
        )
        plsc.subcore_barrier()

        def fire(j, carry):
            pltpu.async_copy(ones_v, acc_sh.at[dst_v.at[j]], sem, add=True)
            return carry

        lax.fori_loop(0, cpt, fire, 0)

        def drain(j, carry):
            pltpu.make_async_copy(ones_v, acc_sh.at[dst_v.at[j]], sem).wait()
            return carry

        lax.fori_loop(0, cpt, drain, 0)
        plsc.subcore_barrier()
        pltpu.sync_copy(
            acc_sh.at[pl.ds(s * rows_per_tile, rows_per_tile)],
            out_hbm.at[c, pl.ds(s * rows_per_tile, rows_per_tile)],
        )

    return k


def _make_scatter_kernel(n_acc, cpt, width, stage_y=False):
    rows_per_tile = n_acc // NS
    nb = cpt // BG
    nbp = nb // 2
    assert cpt % (2 * BG) == 0

    @functools.partial(
        pl.kernel,
        out_type=jax.ShapeDtypeStruct((NC, n_acc, width), jnp.float32),
        mesh=_sc_mesh(),
        compiler_params=_SC_PARAMS,
        scratch_types=[
            pltpu.VMEM((cpt, CH), jnp.int32),
            pltpu.VMEM((cpt, CH), jnp.int32),
            pltpu.VMEM((2, BG * CH, width), jnp.float32),
            pltpu.VMEM_SHARED((n_acc, width), jnp.float32),
        ]
        + ([pltpu.VMEM_SHARED((n_acc, width), jnp.float32)] if stage_y else [])
        + [
            pltpu.SemaphoreType.DMA,
            pltpu.SemaphoreType.DMA,
            pltpu.SemaphoreType.DMA,
            pltpu.SemaphoreType.DMA,
        ],
    )
    def k(y_hbm, src_hbm, dst_hbm, zinit_hbm, out_hbm, src_v, dst_v, rows_v,
          acc_sh, *rest):
        if stage_y:
            y_sh, semg0, semg1, sems0, sems1 = rest
        else:
            semg0, semg1, sems0, sems1 = rest
            y_sh = None
        c = lax.axis_index("c")
        s = lax.axis_index("s")
        w = c * NS + s
        pltpu.sync_copy(src_hbm.at[w], src_v)
        pltpu.sync_copy(dst_hbm.at[w], dst_v)
        if stage_y:
            pltpu.sync_copy(
                y_hbm.at[pl.ds(s * rows_per_tile, rows_per_tile)],
                y_sh.at[pl.ds(s * rows_per_tile, rows_per_tile)],
            )
        y_src = y_sh if stage_y else y_hbm
        pltpu.sync_copy(
            zinit_hbm.at[pl.ds(s * rows_per_tile, rows_per_tile)],
            acc_sh.at[pl.ds(s * rows_per_tile, rows_per_tile)],
        )
        plsc.subcore_barrier()

        semg = (semg0, semg1)
        sems = (sems0, sems1)

        def gslice(h, i):
            return rows_v.at[h, pl.ds(i * CH, CH)]

        def issue_gathers(b, h):
            for i in range(BG):
                pltpu.async_copy(
                    y_src.at[src_v.at[b * BG + i]], gslice(h, i), semg[h]
                )

        def wait_gathers(b, h):
            for i in range(BG):
                pltpu.make_async_copy(
                    y_src.at[src_v.at[b * BG + i]], gslice(h, i), semg[h]
                ).wait()

        def issue_scatters(b, h):
            for i in range(BG):
                pltpu.async_copy(
                    gslice(h, i), acc_sh.at[dst_v.at[b * BG + i]],
                    sems[h], add=True,
                )

        def wait_scatters(b, h):
            for i in range(BG):
                pltpu.make_async_copy(
                    gslice(h, i), acc_sh.at[dst_v.at[b * BG + i]], sems[h]
                ).wait()

        issue_gathers(0, 0)

        def body(bp, carry):
            b0 = 2 * bp
            wait_gathers(b0, 0)
            issue_scatters(b0, 0)

            @pl.when(bp > 0)
            def _():
                wait_scatters(b0 - 1, 1)

            issue_gathers(b0 + 1, 1)
            wait_gathers(b0 + 1, 1)
            issue_scatters(b0 + 1, 1)

            @pl.when(bp < nbp - 1)
            def _():
                wait_scatters(b0, 0)
                issue_gathers(b0 + 2, 0)

            return carry

        lax.fori_loop(0, nbp, body, 0)
        wait_scatters(nb - 2, 0)
        wait_scatters(nb - 1, 1)
        plsc.subcore_barrier()
        pltpu.sync_copy(
            acc_sh.at[pl.ds(s * rows_per_tile, rows_per_tile)],
            out_hbm.at[c, pl.ds(s * rows_per_tile, rows_per_tile)],
        )

    return k


def _make_colsplit_kernel(n_acc, cpt, width):
    rows_per_tile = n_acc // NS
    colw = width // 2
    nb = cpt // BG
    nbp = nb // 2
    assert cpt % (2 * BG) == 0

    @functools.partial(
        pl.kernel,
        out_type=jax.ShapeDtypeStruct((n_acc, width), jnp.float32),
        mesh=_sc_mesh(),
        compiler_params=_SC_PARAMS,
        scratch_types=[
            pltpu.VMEM((cpt, CH), jnp.int32),
            pltpu.VMEM((cpt, CH), jnp.int32),
            pltpu.VMEM((2, BG * CH, colw), jnp.float32),
            pltpu.VMEM_SHARED((n_acc, colw), jnp.float32),
            pltpu.VMEM_SHARED((n_acc, colw), jnp.float32),
            pltpu.SemaphoreType.DMA,
            pltpu.SemaphoreType.DMA,
            pltpu.SemaphoreType.DMA,
            pltpu.SemaphoreType.DMA,
        ],
    )
    def k(y_hbm, src_hbm, dst_hbm, zinit_hbm, out_hbm, src_v, dst_v, rows_v,
          acc_sh, y_sh, semg0, semg1, sems0, sems1):
        c = lax.axis_index("c")
        s = lax.axis_index("s")
        pltpu.sync_copy(src_hbm.at[s], src_v)
        pltpu.sync_copy(dst_hbm.at[s], dst_v)
        pltpu.sync_copy(
            y_hbm.at[pl.ds(s * rows_per_tile, rows_per_tile),
                     pl.ds(c * colw, colw)],
            y_sh.at[pl.ds(s * rows_per_tile, rows_per_tile)],
        )
        pltpu.sync_copy(
            zinit_hbm.at[pl.ds(s * rows_per_tile, rows_per_tile)],
            acc_sh.at[pl.ds(s * rows_per_tile, rows_per_tile)],
        )
        plsc.subcore_barrier()

        semg = (semg0, semg1)
        sems = (sems0, sems1)

        def gslice(h, i):
            return rows_v.at[h, pl.ds(i * CH, CH)]

        def issue_gathers(b, h):
            for i in range(BG):
                pltpu.async_copy(
                    y_sh.at[src_v.at[b * BG + i]], gslice(h, i), semg[h]
                )

        def wait_gathers(b, h):
            for i in range(BG):
                pltpu.make_async_copy(
                    y_sh.at[src_v.at[b * BG + i]], gslice(h, i), semg[h]
                ).wait()

        def issue_scatters(b, h):
            for i in range(BG):
                pltpu.async_copy(
                    gslice(h, i), acc_sh.at[dst_v.at[b * BG + i]],
                    sems[h], add=True,
                )

        def wait_scatters(b, h):
            for i in range(BG):
                pltpu.make_async_copy(
                    gslice(h, i), acc_sh.at[dst_v.at[b * BG + i]], sems[h]
                ).wait()

        issue_gathers(0, 0)

        def body(bp, carry):
            b0 = 2 * bp
            wait_gathers(b0, 0)
            issue_scatters(b0, 0)

            @pl.when(bp > 0)
            def _():
                wait_scatters(b0 - 1, 1)

            issue_gathers(b0 + 1, 1)
            wait_gathers(b0 + 1, 1)
            issue_scatters(b0 + 1, 1)

            @pl.when(bp < nbp - 1)
            def _():
                wait_scatters(b0, 0)
                issue_gathers(b0 + 2, 0)

            return carry

        lax.fori_loop(0, nbp, body, 0)
        wait_scatters(nb - 2, 0)
        wait_scatters(nb - 1, 1)
        plsc.subcore_barrier()
        pltpu.sync_copy(
            acc_sh.at[pl.ds(s * rows_per_tile, rows_per_tile)],
            out_hbm.at[pl.ds(s * rows_per_tile, rows_per_tile),
                       pl.ds(c * colw, colw)],
        )

    return k


def _tc1_body(degacc_ref, x_ref, w1_ref, dinv_ref, y1_ref):
    d = degacc_ref[...]
    deg = d[0, :, 0:1] + d[1, :, 0:1] + 1.0
    dinv = lax.rsqrt(deg)
    xw = jnp.dot(x_ref[...], w1_ref[...], preferred_element_type=jnp.float32)
    y1_ref[...] = xw * dinv
    dinv_ref[...] = jnp.broadcast_to(dinv, dinv_ref.shape)


def _tc2_body(s1_ref, y1_ref, dinv_ref, w2_ref, b1_ref, y2_ref):
    dinv = dinv_ref[...][:, 0:1]
    h = (s1_ref[...] + y1_ref[...]) * dinv + b1_ref[...]
    h = jnp.maximum(h, 0.0)
    hw = jnp.dot(h, w2_ref[...], preferred_element_type=jnp.float32)
    y2_ref[...] = hw * dinv


def _tc3_body(s2_ref, y2_ref, dinv_ref, b2_ref, out_ref):
    s2 = s2_ref[...]
    dinv = dinv_ref[...][:, 0:1]
    z = (s2[0] + s2[1] + y2_ref[...]) * dinv + b2_ref[...]
    m = jnp.max(z, axis=1, keepdims=True)
    e = jnp.exp(z - m)
    out_ref[...] = z - m - jnp.log(jnp.sum(e, axis=1, keepdims=True))


def kernel(x, edge_index, W1, b1, W2, b2):
    n, d_in = x.shape
    e = edge_index.shape[1]
    h_dim = W1.shape[1]
    c_dim = W2.shape[1]

    n_acc = ((n + 511) // 512 + 1) * 512
    blk = 512
    n_blocks = n_acc // blk

    n_dummy = n_acc - n

    def edge_layout(n_slices):
        ept = -(-e // n_slices)
        cpt = -(-ept // (CH * 2 * BG)) * 2 * BG
        e_pad = n_slices * cpt * CH
        pad_dst = n + jnp.arange(e_pad - e, dtype=jnp.int32) % n_dummy
        src_p = jnp.concatenate(
            [edge_index[0], jnp.zeros((e_pad - e,), jnp.int32)]
        ).reshape(n_slices, cpt, CH)
        dst_p = jnp.concatenate([edge_index[1], pad_dst]).reshape(
            n_slices, cpt, CH
        )
        return src_p, dst_p, cpt

    src_p, dst_p, cpt = edge_layout(NW)
    src16, dst16, cpt16 = edge_layout(NS)

    dw = 16
    ones_buf = jnp.ones((CH, dw), jnp.float32)
    zin_dw = jnp.zeros((n_acc, dw), jnp.float32)
    zin_h = jnp.zeros((n_acc, h_dim), jnp.float32)
    zin_c = jnp.zeros((n_acc, c_dim), jnp.float32)
    x_pad = jnp.concatenate([x, jnp.zeros((n_acc - n, d_in), x.dtype)])

    degacc = _make_deg_kernel(n_acc, cpt, dw)(dst_p, ones_buf, zin_dw)

    dinv, y1 = pl.pallas_call(
        _tc1_body,
        grid=(n_blocks,),
        in_specs=[
            pl.BlockSpec((NC, blk, dw), lambda i: (0, i, 0)),
            pl.BlockSpec((blk, d_in), lambda i: (i, 0)),
            pl.BlockSpec((d_in, h_dim), lambda i: (0, 0)),
        ],
        out_specs=[
            pl.BlockSpec((blk, 8), lambda i: (i, 0)),
            pl.BlockSpec((blk, h_dim), lambda i: (i, 0)),
        ],
        out_shape=[
            jax.ShapeDtypeStruct((n_acc, 8), jnp.float32),
            jax.ShapeDtypeStruct((n_acc, h_dim), jnp.float32),
        ],
    )(degacc, x_pad, W1)

    zin_h2 = jnp.zeros((n_acc, h_dim // 2), jnp.float32)
    s1 = _make_colsplit_kernel(n_acc, cpt16, h_dim)(
        y1, src16, dst16, zin_h2
    )

    y2 = pl.pallas_call(
        _tc2_body,
        grid=(n_blocks,),
        in_specs=[
            pl.BlockSpec((blk, h_dim), lambda i: (i, 0)),
            pl.BlockSpec((blk, h_dim), lambda i: (i, 0)),
            pl.BlockSpec((blk, 8), lambda i: (i, 0)),
            pl.BlockSpec((h_dim, c_dim), lambda i: (0, 0)),
            pl.BlockSpec((1, h_dim), lambda i: (0, 0)),
        ],
        out_specs=pl.BlockSpec((blk, c_dim), lambda i: (i, 0)),
        out_shape=jax.ShapeDtypeStruct((n_acc, c_dim), jnp.float32),
    )(s1, y1, dinv, W2, b1.reshape(1, h_dim))

    s2 = _make_scatter_kernel(n_acc, cpt, c_dim, stage_y=True)(
        y2, src_p, dst_p, zin_c
    )

    out = pl.pallas_call(
        _tc3_body,
        grid=(n_blocks,),
        in_specs=[
            pl.BlockSpec((NC, blk, c_dim), lambda i: (0, i, 0)),
            pl.BlockSpec((blk, c_dim), lambda i: (i, 0)),
            pl.BlockSpec((blk, 8), lambda i: (i, 0)),
            pl.BlockSpec((1, c_dim), lambda i: (0, 0)),
        ],
        out_specs=pl.BlockSpec((blk, c_dim), lambda i: (i, 0)),
        out_shape=jax.ShapeDtypeStruct((n_acc, c_dim), jnp.float32),
    )(s2, y2, dinv, b2.reshape(1, c_dim))

    return out[:n]


# speedup vs baseline: 41.9413x; 1.0593x over previous
import functools

import jax
import jax.numpy as jnp
from jax import lax
from jax.experimental import pallas as pl
from jax.experimental.pallas import tpu as pltpu
from jax.experimental.pallas import tpu_sc as plsc

NC = 2
NS = 16
NW = NC * NS

CH = 128
BG = 4


def _sc_mesh():
    return plsc.VectorSubcoreMesh(
        core_axis_name="c", subcore_axis_name="s", num_cores=NC, num_subcores=NS
    )


_SC_PARAMS = pltpu.CompilerParams(use_tc_tiling_on_sc=False)


def _make_deg_kernel(n_acc, cpt, dw):
    rows_per_tile = n_acc // NS

    @functools.partial(
        pl.kernel,
        out_type=jax.ShapeDtypeStruct((NC, n_acc, dw), jnp.float32),
        mesh=_sc_mesh(),
        compiler_params=_SC_PARAMS,
        scratch_types=[
            pltpu.VMEM((cpt, CH), jnp.int32),
            pltpu.VMEM((CH, dw), jnp.float32),
            pltpu.VMEM_SHARED((n_acc, dw), jnp.float32),
            pltpu.SemaphoreType.DMA,
        ],
    )
    def k(dst_hbm, ones_hbm, zinit_hbm, out_hbm, dst_v, ones_v, acc_sh, sem):
        c = lax.axis_index("c")
        s = lax.axis_index("s")
        w = c * NS + s
        pltpu.sync_copy(dst_hbm.at[w], dst_v)
        pltpu.sync_copy(ones_hbm, ones_v)
        pltpu.sync_copy(
            zinit_hbm.at[pl.ds(s * rows_per_tile, rows_per_tile)],
            acc_sh.at[pl.ds(s * rows_per_tile, rows_per_tile)],
        )
        plsc.subcore_barrier()

        def fire(j, carry):
            pltpu.async_copy(ones_v, acc_sh.at[dst_v.at[j]], sem, add=True)
            return carry

        lax.fori_loop(0, cpt, fire, 0)

        def drain(j, carry):
            pltpu.make_async_copy(ones_v, acc_sh.at[dst_v.at[j]], sem).wait()
            return carry

        lax.fori_loop(0, cpt, drain, 0)
        plsc.subcore_barrier()
        pltpu.sync_copy(
            acc_sh.at[pl.ds(s * rows_per_tile, rows_per_tile)],
            out_hbm.at[c, pl.ds(s * rows_per_tile, rows_per_tile)],
        )

    return k


def _make_scatter_kernel(n_acc, cpt, width, stage_y=False):
    rows_per_tile = n_acc // NS
    nb = cpt // BG
    nbp = nb // 2
    assert cpt % (2 * BG) == 0

    @functools.partial(
        pl.kernel,
        out_type=jax.ShapeDtypeStruct((NC, n_acc, width), jnp.float32),
        mesh=_sc_mesh(),
        compiler_params=_SC_PARAMS,
        scratch_types=[
            pltpu.VMEM((cpt, CH), jnp.int32),
            pltpu.VMEM((cpt, CH), jnp.int32),
            pltpu.VMEM((2, BG * CH, width), jnp.float32),
            pltpu.VMEM_SHARED((n_acc, width), jnp.float32),
        ]
        + ([pltpu.VMEM_SHARED((n_acc, width), jnp.float32)] if stage_y else [])
        + [
            pltpu.SemaphoreType.DMA,
            pltpu.SemaphoreType.DMA,
            pltpu.SemaphoreType.DMA,
            pltpu.SemaphoreType.DMA,
        ],
    )
    def k(y_hbm, src_hbm, dst_hbm, zinit_hbm, out_hbm, src_v, dst_v, rows_v,
          acc_sh, *rest):
        if stage_y:
            y_sh, semg0, semg1, sems0, sems1 = rest
        else:
            semg0, semg1, sems0, sems1 = rest
            y_sh = None
        c = lax.axis_index("c")
        s = lax.axis_index("s")
        w = c * NS + s
        pltpu.sync_copy(src_hbm.at[w], src_v)
        pltpu.sync_copy(dst_hbm.at[w], dst_v)
        if stage_y:
            pltpu.sync_copy(
                y_hbm.at[pl.ds(s * rows_per_tile, rows_per_tile)],
                y_sh.at[pl.ds(s * rows_per_tile, rows_per_tile)],
            )
        y_src = y_sh if stage_y else y_hbm
        pltpu.sync_copy(
            zinit_hbm.at[pl.ds(s * rows_per_tile, rows_per_tile)],
            acc_sh.at[pl.ds(s * rows_per_tile, rows_per_tile)],
        )
        plsc.subcore_barrier()

        semg = (semg0, semg1)
        sems = (sems0, sems1)

        def gslice(h, i):
            return rows_v.at[h, pl.ds(i * CH, CH)]

        def issue_gathers(b, h):
            for i in range(BG):
                pltpu.async_copy(
                    y_src.at[src_v.at[b * BG + i]], gslice(h, i), semg[h]
                )

        def wait_gathers(b, h):
            for i in range(BG):
                pltpu.make_async_copy(
                    y_src.at[src_v.at[b * BG + i]], gslice(h, i), semg[h]
                ).wait()

        def issue_scatters(b, h):
            for i in range(BG):
                pltpu.async_copy(
                    gslice(h, i), acc_sh.at[dst_v.at[b * BG + i]],
                    sems[h], add=True,
                )

        def wait_scatters(b, h):
            for i in range(BG):
                pltpu.make_async_copy(
                    gslice(h, i), acc_sh.at[dst_v.at[b * BG + i]], sems[h]
                ).wait()

        issue_gathers(0, 0)

        def body(bp, carry):
            b0 = 2 * bp
            wait_gathers(b0, 0)
            issue_scatters(b0, 0)

            @pl.when(bp > 0)
            def _():
                wait_scatters(b0 - 1, 1)

            issue_gathers(b0 + 1, 1)
            wait_gathers(b0 + 1, 1)
            issue_scatters(b0 + 1, 1)

            @pl.when(bp < nbp - 1)
            def _():
                wait_scatters(b0, 0)
                issue_gathers(b0 + 2, 0)

            return carry

        lax.fori_loop(0, nbp, body, 0)
        wait_scatters(nb - 2, 0)
        wait_scatters(nb - 1, 1)
        plsc.subcore_barrier()
        pltpu.sync_copy(
            acc_sh.at[pl.ds(s * rows_per_tile, rows_per_tile)],
            out_hbm.at[c, pl.ds(s * rows_per_tile, rows_per_tile)],
        )

    return k


def _make_colsplit_kernel(n_acc, cpt32, width):
    rows_per_tile = n_acc // NS
    colw = width // 2
    cpt = 2 * cpt32
    nb = cpt // BG
    nbp = nb // 2
    assert cpt % (2 * BG) == 0

    @functools.partial(
        pl.kernel,
        out_type=jax.ShapeDtypeStruct((n_acc, width), jnp.float32),
        mesh=_sc_mesh(),
        compiler_params=_SC_PARAMS,
        scratch_types=[
            pltpu.VMEM((cpt, CH), jnp.int32),
            pltpu.VMEM((cpt, CH), jnp.int32),
            pltpu.VMEM((2, BG * CH, colw), jnp.float32),
            pltpu.VMEM_SHARED((n_acc, colw), jnp.float32),
            pltpu.VMEM_SHARED((n_acc, colw), jnp.float32),
            pltpu.SemaphoreType.DMA,
            pltpu.SemaphoreType.DMA,
            pltpu.SemaphoreType.DMA,
            pltpu.SemaphoreType.DMA,
        ],
    )
    def k(y_hbm, src_hbm, dst_hbm, zinit_hbm, out_hbm, src_v, dst_v, rows_v,
          acc_sh, y_sh, semg0, semg1, sems0, sems1):
        c = lax.axis_index("c")
        s = lax.axis_index("s")
        half = cpt // 2
        pltpu.sync_copy(src_hbm.at[2 * s], src_v.at[pl.ds(0, half)])
        pltpu.sync_copy(src_hbm.at[2 * s + 1], src_v.at[pl.ds(half, half)])
        pltpu.sync_copy(dst_hbm.at[2 * s], dst_v.at[pl.ds(0, half)])
        pltpu.sync_copy(dst_hbm.at[2 * s + 1], dst_v.at[pl.ds(half, half)])
        pltpu.sync_copy(
            y_hbm.at[pl.ds(s * rows_per_tile, rows_per_tile),
                     pl.ds(c * colw, colw)],
            y_sh.at[pl.ds(s * rows_per_tile, rows_per_tile)],
        )
        pltpu.sync_copy(
            zinit_hbm.at[pl.ds(s * rows_per_tile, rows_per_tile)],
            acc_sh.at[pl.ds(s * rows_per_tile, rows_per_tile)],
        )
        plsc.subcore_barrier()

        semg = (semg0, semg1)
        sems = (sems0, sems1)

        def gslice(h, i):
            return rows_v.at[h, pl.ds(i * CH, CH)]

        def issue_gathers(b, h):
            for i in range(BG):
                pltpu.async_copy(
                    y_sh.at[src_v.at[b * BG + i]], gslice(h, i), semg[h]
                )

        def wait_gathers(b, h):
            for i in range(BG):
                pltpu.make_async_copy(
                    y_sh.at[src_v.at[b * BG + i]], gslice(h, i), semg[h]
                ).wait()

        def issue_scatters(b, h):
            for i in range(BG):
                pltpu.async_copy(
                    gslice(h, i), acc_sh.at[dst_v.at[b * BG + i]],
                    sems[h], add=True,
                )

        def wait_scatters(b, h):
            for i in range(BG):
                pltpu.make_async_copy(
                    gslice(h, i), acc_sh.at[dst_v.at[b * BG + i]], sems[h]
                ).wait()

        issue_gathers(0, 0)

        def body(bp, carry):
            b0 = 2 * bp
            wait_gathers(b0, 0)
            issue_scatters(b0, 0)

            @pl.when(bp > 0)
            def _():
                wait_scatters(b0 - 1, 1)

            issue_gathers(b0 + 1, 1)
            wait_gathers(b0 + 1, 1)
            issue_scatters(b0 + 1, 1)

            @pl.when(bp < nbp - 1)
            def _():
                wait_scatters(b0, 0)
                issue_gathers(b0 + 2, 0)

            return carry

        lax.fori_loop(0, nbp, body, 0)
        wait_scatters(nb - 2, 0)
        wait_scatters(nb - 1, 1)
        plsc.subcore_barrier()
        pltpu.sync_copy(
            acc_sh.at[pl.ds(s * rows_per_tile, rows_per_tile)],
            out_hbm.at[pl.ds(s * rows_per_tile, rows_per_tile),
                       pl.ds(c * colw, colw)],
        )

    return k


def _tc1a_body(x_ref, w1_ref, xw_ref):
    xw_ref[...] = jnp.dot(
        x_ref[...], w1_ref[...], preferred_element_type=jnp.float32
    )


def _tc1b_body(degacc_ref, xw_ref, dinv_ref, y1_ref):
    d = degacc_ref[...]
    deg = d[0, :, 0:1] + d[1, :, 0:1] + 1.0
    dinv = lax.rsqrt(deg)
    y1_ref[...] = xw_ref[...] * dinv
    dinv_ref[...] = jnp.broadcast_to(dinv, dinv_ref.shape)


def _tc2_body(s1_ref, y1_ref, dinv_ref, w2_ref, b1_ref, y2_ref):
    dinv = dinv_ref[...][:, 0:1]
    h = (s1_ref[...] + y1_ref[...]) * dinv + b1_ref[...]
    h = jnp.maximum(h, 0.0)
    hw = jnp.dot(h, w2_ref[...], preferred_element_type=jnp.float32)
    y2_ref[...] = hw * dinv


def _tc3_body(s2_ref, y2_ref, dinv_ref, b2_ref, out_ref):
    s2 = s2_ref[...]
    dinv = dinv_ref[...][:, 0:1]
    z = (s2[0] + s2[1] + y2_ref[...]) * dinv + b2_ref[...]
    m = jnp.max(z, axis=1, keepdims=True)
    e = jnp.exp(z - m)
    out_ref[...] = z - m - jnp.log(jnp.sum(e, axis=1, keepdims=True))


def kernel(x, edge_index, W1, b1, W2, b2):
    n, d_in = x.shape
    e = edge_index.shape[1]
    h_dim = W1.shape[1]
    c_dim = W2.shape[1]

    n_acc = ((n + 511) // 512 + 1) * 512
    blk = 512
    n_blocks = n_acc // blk

    n_dummy = n_acc - n

    def edge_layout(n_slices):
        ept = -(-e // n_slices)
        cpt = -(-ept // (CH * 2 * BG)) * 2 * BG
        e_pad = n_slices * cpt * CH
        pad_dst = n + jnp.arange(e_pad - e, dtype=jnp.int32) % n_dummy
        src_p = jnp.concatenate(
            [edge_index[0], jnp.zeros((e_pad - e,), jnp.int32)]
        ).reshape(n_slices, cpt, CH)
        dst_p = jnp.concatenate([edge_index[1], pad_dst]).reshape(
            n_slices, cpt, CH
        )
        return src_p, dst_p, cpt

    src_p, dst_p, cpt = edge_layout(NW)

    dw = 16
    ones_buf = jnp.ones((CH, dw), jnp.float32)
    zin_dw = jnp.zeros((n_acc, dw), jnp.float32)
    zin_c = jnp.zeros((n_acc, c_dim), jnp.float32)
    zin_h2 = jnp.zeros((n_acc, h_dim // 2), jnp.float32)
    x_pad = jnp.concatenate([x, jnp.zeros((n_acc - n, d_in), x.dtype)])

    degacc = _make_deg_kernel(n_acc, cpt, dw)(dst_p, ones_buf, zin_dw)
    xw = pl.pallas_call(
        _tc1a_body,
        out_shape=jax.ShapeDtypeStruct((n_acc, h_dim), jnp.float32),
    )(x_pad, W1)

    dinv, y1 = pl.pallas_call(
        _tc1b_body,
        out_shape=[
            jax.ShapeDtypeStruct((n_acc, 8), jnp.float32),
            jax.ShapeDtypeStruct((n_acc, h_dim), jnp.float32),
        ],
    )(degacc, xw)

    s1 = _make_colsplit_kernel(n_acc, cpt, h_dim)(y1, src_p, dst_p, zin_h2)

    y2 = pl.pallas_call(
        _tc2_body,
        out_shape=jax.ShapeDtypeStruct((n_acc, c_dim), jnp.float32),
    )(s1, y1, dinv, W2, b1.reshape(1, h_dim))

    s2 = _make_scatter_kernel(n_acc, cpt, c_dim, stage_y=True)(
        y2, src_p, dst_p, zin_c
    )

    out = pl.pallas_call(
        _tc3_body,
        grid=(n_blocks,),
        in_specs=[
            pl.BlockSpec((NC, blk, c_dim), lambda i: (0, i, 0)),
            pl.BlockSpec((blk, c_dim), lambda i: (i, 0)),
            pl.BlockSpec((blk, 8), lambda i: (i, 0)),
            pl.BlockSpec((1, c_dim), lambda i: (0, 0)),
        ],
        out_specs=pl.BlockSpec((blk, c_dim), lambda i: (i, 0)),
        out_shape=jax.ShapeDtypeStruct((n_acc, c_dim), jnp.float32),
    )(s2, y2, dinv, b2.reshape(1, c_dim))

    return out[:n]
